# baseline (speedup 1.0000x reference)
import functools
import math

import jax
import jax.numpy as jnp
from jax import lax
from jax.experimental import pallas as pl
from jax.experimental.pallas import tpu as pltpu
from jax.experimental.pallas import tpu_sc as plsc

B, C, T, N, H = 16, 512, 576, 1024, 4
DH = C // H
SF = 1.0 / math.sqrt(DH)
INV_SQRT_H = 1.0 / math.sqrt(H)

_NC = 2
_NS = 16
_L = 16
_NW = _NC * _NS
_ROWS = C // _NW
_NG = T // _L


def _bf16_round(x):
    return lax.convert_element_type(
        lax.convert_element_type(x, jnp.bfloat16), jnp.float32)


def _prep_body(cb_ref, wk_ref, bk_ref, wv_ref, bv_ref, kf_ref, vt_ref):
    cb = cb_ref[...]
    kf_ref[...] = lax.dot_general(
        cb, wk_ref[...], (((1,), (1,)), ((), ())),
        preferred_element_type=jnp.float32) + bk_ref[...]
    val = lax.dot_general(
        cb, wv_ref[...], (((1,), (1,)), ((), ())),
        preferred_element_type=jnp.float32) + bv_ref[...]
    vt_ref[...] = jnp.transpose(_bf16_round(val))


def _main_body(x_ref, wq_ref, bq_ref, wp_ref, bp_ref, kf_ref,
               logits_ref, idx_ref):
    x = x_ref[0]
    q = lax.dot_general(x, wq_ref[...], (((0,), (1,)), ((), ())),
                        preferred_element_type=jnp.float32)
    q = q + bq_ref[...]
    cp = lax.dot_general(x, wp_ref[...], (((0,), (1,)), ((), ())),
                         preferred_element_type=jnp.float32)
    cp = jnp.transpose(cp + bp_ref[...])
    cpb = _bf16_round(cp)
    kf = kf_ref[...]
    acc = None
    for h in range(H):
        q_h = q[:, h * DH:(h + 1) * DH]
        k_h = kf[:, h * DH:(h + 1) * DH]
        l1 = lax.dot_general(k_h, q_h, (((1,), (1,)), ((), ())),
                             preferred_element_type=jnp.float32) * SF
        term = cpb[h:h + 1, :] * _bf16_round(l1)
        acc = term if acc is None else acc + term
    logits = acc * INV_SQRT_H
    logits_ref[0] = logits
    maxv = jnp.max(logits, axis=0, keepdims=True)
    iota = lax.broadcasted_iota(jnp.int32, (N, T), 0)
    cand = jnp.where(logits == maxv, iota, N)
    idx = jnp.min(cand, axis=0, keepdims=True)
    idx_ref[0] = idx


def _zq_gather_body(vt_hbm, idx_hbm, zq_hbm, tbl_v, idx_v, out_v):
    wid = lax.axis_index("s") * _NC + lax.axis_index("c")
    c0 = wid * _ROWS
    pltpu.sync_copy(vt_hbm.at[pl.ds(c0 * N, _ROWS * N)], tbl_v)
    for b in range(B):
        pltpu.sync_copy(idx_hbm.at[pl.ds(b * T, T)], idx_v)

        def body(g, carry):
            ids = idx_v[pl.ds(g * _L, _L)]
            for c in range(_ROWS):
                out_v[pl.ds(c * T + g * _L, _L)] = plsc.load_gather(
                    tbl_v, [ids + (c * N)])
            return carry

        lax.fori_loop(0, _NG, body, 0)
        pltpu.sync_copy(out_v, zq_hbm.at[pl.ds((b * C + c0) * T, _ROWS * T)])


def _zq_gather(value_t, idx2d):
    gather = functools.partial(
        pl.kernel,
        mesh=plsc.VectorSubcoreMesh(core_axis_name="c",
                                    subcore_axis_name="s"),
        compiler_params=pltpu.CompilerParams(needs_layout_passes=False),
        out_type=jax.ShapeDtypeStruct((B * C * T,), jnp.float32),
        scratch_types=[
            pltpu.VMEM((_ROWS * N,), jnp.float32),
            pltpu.VMEM((T,), jnp.int32),
            pltpu.VMEM((_ROWS * T,), jnp.float32),
        ],
    )(_zq_gather_body)
    return gather(value_t.reshape(C * N), idx2d.reshape(B * T)).reshape(
        B, C, T)


def kernel(hidden_states, codebook_hidden_states, Wq, bq, Wk, bk, Wv, bv,
           Wp, bp):
    bk2 = bk.reshape(1, C)
    bv2 = bv.reshape(1, C)
    bq2 = bq.reshape(1, C)
    bp2 = bp.reshape(1, H)

    keyflat, value_t = pl.pallas_call(
        _prep_body,
        out_shape=(
            jax.ShapeDtypeStruct((N, C), jnp.float32),
            jax.ShapeDtypeStruct((C, N), jnp.float32),
        ),
    )(codebook_hidden_states, Wk, bk2, Wv, bv2)

    full = lambda shape: pl.BlockSpec(shape, lambda b: (0,) * len(shape))
    logits, idx = pl.pallas_call(
        _main_body,
        grid=(B,),
        in_specs=[
            pl.BlockSpec((1, C, T), lambda b: (b, 0, 0)),
            full((C, C)),
            full((1, C)),
            full((H, C)),
            full((1, H)),
            full((N, C)),
        ],
        out_specs=(
            pl.BlockSpec((1, N, T), lambda b: (b, 0, 0)),
            pl.BlockSpec((1, 1, T), lambda b: (b, 0, 0)),
        ),
        out_shape=(
            jax.ShapeDtypeStruct((B, N, T), jnp.float32),
            jax.ShapeDtypeStruct((B, 1, T), jnp.int32),
        ),
    )(hidden_states, Wq, bq2, Wp, bp2, keyflat)

    zq = _zq_gather(value_t, idx.reshape(B, T))

    return (logits, idx, zq)


# speedup vs baseline: 1.0042x; 1.0042x over previous
import functools
import math

import jax
import jax.numpy as jnp
from jax import lax
from jax.experimental import pallas as pl
from jax.experimental.pallas import tpu as pltpu
from jax.experimental.pallas import tpu_sc as plsc

B, C, T, N, H = 16, 512, 576, 1024, 4
DH = C // H
SF = 1.0 / math.sqrt(DH)
INV_SQRT_H = 1.0 / math.sqrt(H)

_NC = 2
_NS = 16
_L = 16
_NW = _NC * _NS
_ROWS = C // _NW
_NG = T // _L


def _bf16_round(x):
    return lax.convert_element_type(
        lax.convert_element_type(x, jnp.bfloat16), jnp.float32)


def _prep_body(cb_ref, wk_ref, bk_ref, wv_ref, bv_ref, kf_ref, vt_ref):
    cb = cb_ref[...]
    kf_ref[...] = lax.dot_general(
        cb, wk_ref[...], (((1,), (1,)), ((), ())),
        preferred_element_type=jnp.float32) + bk_ref[...]
    val = lax.dot_general(
        cb, wv_ref[...], (((1,), (1,)), ((), ())),
        preferred_element_type=jnp.float32) + bv_ref[...]
    vt_ref[...] = jnp.transpose(_bf16_round(val))


def _main_body(x_ref, wq_ref, bq_ref, wp_ref, bp_ref, kf_ref,
               logits_ref, idx_ref):
    x = x_ref[0]
    q = lax.dot_general(x, wq_ref[...], (((0,), (1,)), ((), ())),
                        preferred_element_type=jnp.float32)
    q = q + bq_ref[...]
    cp = lax.dot_general(x, wp_ref[...], (((0,), (1,)), ((), ())),
                         preferred_element_type=jnp.float32)
    cp = jnp.transpose(cp + bp_ref[...])
    cpb = _bf16_round(cp)
    kf = kf_ref[...]
    acc = None
    for h in range(H):
        q_h = q[:, h * DH:(h + 1) * DH]
        k_h = kf[:, h * DH:(h + 1) * DH]
        l1 = lax.dot_general(k_h, q_h, (((1,), (1,)), ((), ())),
                             preferred_element_type=jnp.float32) * SF
        term = cpb[h:h + 1, :] * _bf16_round(l1)
        acc = term if acc is None else acc + term
    logits = acc * INV_SQRT_H
    logits_ref[0] = logits
    maxv = jnp.max(logits, axis=0, keepdims=True)
    iota = lax.broadcasted_iota(jnp.int32, (N, T), 0)
    cand = jnp.where(logits == maxv, iota, N)
    idx = jnp.min(cand, axis=0, keepdims=True)
    idx_ref[0] = idx


def _zq_gather_body(vt_hbm, idx_hbm, zq_hbm, tbl_v, idx_v, out_v):
    wid = lax.axis_index("s") * _NC + lax.axis_index("c")
    c0 = wid * _ROWS
    pltpu.sync_copy(vt_hbm.at[pl.ds(c0, _ROWS)], tbl_v)
    for b in range(B):
        pltpu.sync_copy(idx_hbm.at[b], idx_v)

        def body(g, carry):
            ids = idx_v[pl.ds(g * _L, _L)]
            for c in range(_ROWS):
                row = jnp.full((_L,), c, jnp.int32)
                out_v[c, pl.ds(g * _L, _L)] = plsc.load_gather(
                    tbl_v, [row, ids])
            return carry

        lax.fori_loop(0, _NG, body, 0)
        pltpu.sync_copy(out_v, zq_hbm.at[b, pl.ds(c0, _ROWS)])


def _zq_gather(value_t, idx2d):
    gather = functools.partial(
        pl.kernel,
        mesh=plsc.VectorSubcoreMesh(core_axis_name="c",
                                    subcore_axis_name="s"),
        compiler_params=pltpu.CompilerParams(
            use_tc_tiling_on_sc=False, needs_layout_passes=False),
        out_type=jax.ShapeDtypeStruct((B, C, T), jnp.float32),
        scratch_types=[
            pltpu.VMEM((_ROWS, N), jnp.float32),
            pltpu.VMEM((T,), jnp.int32),
            pltpu.VMEM((_ROWS, T), jnp.float32),
        ],
    )(_zq_gather_body)
    return gather(value_t, idx2d)


def kernel(hidden_states, codebook_hidden_states, Wq, bq, Wk, bk, Wv, bv,
           Wp, bp):
    bk2 = bk.reshape(1, C)
    bv2 = bv.reshape(1, C)
    bq2 = bq.reshape(1, C)
    bp2 = bp.reshape(1, H)

    keyflat, value_t = pl.pallas_call(
        _prep_body,
        out_shape=(
            jax.ShapeDtypeStruct((N, C), jnp.float32),
            jax.ShapeDtypeStruct((C, N), jnp.float32),
        ),
    )(codebook_hidden_states, Wk, bk2, Wv, bv2)

    full = lambda shape: pl.BlockSpec(shape, lambda b: (0,) * len(shape))
    logits, idx = pl.pallas_call(
        _main_body,
        grid=(B,),
        in_specs=[
            pl.BlockSpec((1, C, T), lambda b: (b, 0, 0)),
            full((C, C)),
            full((1, C)),
            full((H, C)),
            full((1, H)),
            full((N, C)),
        ],
        out_specs=(
            pl.BlockSpec((1, N, T), lambda b: (b, 0, 0)),
            pl.BlockSpec((1, 1, T), lambda b: (b, 0, 0)),
        ),
        out_shape=(
            jax.ShapeDtypeStruct((B, N, T), jnp.float32),
            jax.ShapeDtypeStruct((B, 1, T), jnp.int32),
        ),
    )(hidden_states, Wq, bq2, Wp, bp2, keyflat)

    zq = _zq_gather(value_t, idx.reshape(B, T))

    return (logits, idx, zq)


# speedup vs baseline: 1.1915x; 1.1865x over previous
import functools
import math

import jax
import jax.numpy as jnp
from jax import lax
from jax.experimental import pallas as pl
from jax.experimental.pallas import tpu as pltpu
from jax.experimental.pallas import tpu_sc as plsc

B, C, T, N, H = 16, 512, 576, 1024, 4
DH = C // H
SF = 1.0 / math.sqrt(DH)
INV_SQRT_H = 1.0 / math.sqrt(H)

_NC = 2
_NS = 16
_L = 16
_NW = _NC * _NS
_ROWS = C // _NW
_NG = T // _L


def _bf16_round(x):
    return lax.convert_element_type(
        lax.convert_element_type(x, jnp.bfloat16), jnp.float32)


def _prep_body(cb_ref, wk_ref, bk_ref, wv_ref, bv_ref, kf_ref, vt_ref):
    cb = cb_ref[...]
    kf_ref[...] = lax.dot_general(
        cb, wk_ref[...], (((1,), (1,)), ((), ())),
        preferred_element_type=jnp.float32) + bk_ref[...]
    val = lax.dot_general(
        cb, wv_ref[...], (((1,), (1,)), ((), ())),
        preferred_element_type=jnp.float32) + bv_ref[...]
    vt_ref[...] = jnp.transpose(_bf16_round(val))


def _main_body(x_ref, wq_ref, bq_ref, wp_ref, bp_ref, kf_ref,
               logits_ref, idx_ref):
    x = x_ref[0]
    q = lax.dot_general(x, wq_ref[...], (((0,), (1,)), ((), ())),
                        preferred_element_type=jnp.float32)
    q = q + bq_ref[...]
    cp = lax.dot_general(x, wp_ref[...], (((0,), (1,)), ((), ())),
                         preferred_element_type=jnp.float32)
    cp = jnp.transpose(cp + bp_ref[...])
    cpb = _bf16_round(cp)
    kf = kf_ref[...]
    acc = None
    for h in range(H):
        q_h = q[:, h * DH:(h + 1) * DH]
        k_h = kf[:, h * DH:(h + 1) * DH]
        l1 = lax.dot_general(k_h, q_h, (((1,), (1,)), ((), ())),
                             preferred_element_type=jnp.float32) * SF
        term = cpb[h:h + 1, :] * _bf16_round(l1)
        acc = term if acc is None else acc + term
    logits = acc * INV_SQRT_H
    logits_ref[0] = logits
    maxv = jnp.max(logits, axis=0, keepdims=True)
    iota = lax.broadcasted_iota(jnp.int32, (N, T), 0)
    cand = jnp.where(logits == maxv, iota, N)
    idx = jnp.min(cand, axis=0, keepdims=True)
    idx_ref[0] = idx


def _zq_gather_body(vt_hbm, idx_hbm, zq_hbm, tbl_v, idx_v, out_v,
                    isem0, isem1, osem0, osem1):
    wid = lax.axis_index("s") * _NC + lax.axis_index("c")
    c0 = wid * _ROWS
    isems = (isem0, isem1)
    osems = (osem0, osem1)
    pltpu.sync_copy(vt_hbm.at[pl.ds(c0, _ROWS)], tbl_v)
    pltpu.async_copy(idx_hbm.at[0], idx_v.at[0], isems[0]).wait()
    out_cp = [None, None]
    for b in range(B):
        cur = b % 2
        nxt = (b + 1) % 2
        next_idx_cp = None
        if b + 1 < B:
            next_idx_cp = pltpu.async_copy(
                idx_hbm.at[b + 1], idx_v.at[nxt], isems[nxt])
        if out_cp[cur] is not None:
            out_cp[cur].wait()

        @plsc.parallel_loop(0, _NG, unroll=4)
        def gbody(g):
            ids = idx_v[cur, pl.ds(g * _L, _L)]
            for c in range(_ROWS):
                row = jnp.full((_L,), c, jnp.int32)
                out_v[cur, c, pl.ds(g * _L, _L)] = plsc.load_gather(
                    tbl_v, [row, ids])

        out_cp[cur] = pltpu.async_copy(
            out_v.at[cur], zq_hbm.at[b, pl.ds(c0, _ROWS)], osems[cur])
        if next_idx_cp is not None:
            next_idx_cp.wait()
    out_cp[0].wait()
    out_cp[1].wait()


def _zq_gather(value_t, idx2d):
    gather = functools.partial(
        pl.kernel,
        mesh=plsc.VectorSubcoreMesh(core_axis_name="c",
                                    subcore_axis_name="s"),
        compiler_params=pltpu.CompilerParams(
            use_tc_tiling_on_sc=False, needs_layout_passes=False),
        out_type=jax.ShapeDtypeStruct((B, C, T), jnp.float32),
        scratch_types=[
            pltpu.VMEM((_ROWS, N), jnp.float32),
            pltpu.VMEM((2, T), jnp.int32),
            pltpu.VMEM((2, _ROWS, T), jnp.float32),
            pltpu.SemaphoreType.DMA,
            pltpu.SemaphoreType.DMA,
            pltpu.SemaphoreType.DMA,
            pltpu.SemaphoreType.DMA,
        ],
    )(_zq_gather_body)
    return gather(value_t, idx2d)


def kernel(hidden_states, codebook_hidden_states, Wq, bq, Wk, bk, Wv, bv,
           Wp, bp):
    bk2 = bk.reshape(1, C)
    bv2 = bv.reshape(1, C)
    bq2 = bq.reshape(1, C)
    bp2 = bp.reshape(1, H)

    keyflat, value_t = pl.pallas_call(
        _prep_body,
        out_shape=(
            jax.ShapeDtypeStruct((N, C), jnp.float32),
            jax.ShapeDtypeStruct((C, N), jnp.float32),
        ),
    )(codebook_hidden_states, Wk, bk2, Wv, bv2)

    full = lambda shape: pl.BlockSpec(shape, lambda b: (0,) * len(shape))
    logits, idx = pl.pallas_call(
        _main_body,
        grid=(B,),
        in_specs=[
            pl.BlockSpec((1, C, T), lambda b: (b, 0, 0)),
            full((C, C)),
            full((1, C)),
            full((H, C)),
            full((1, H)),
            full((N, C)),
        ],
        out_specs=(
            pl.BlockSpec((1, N, T), lambda b: (b, 0, 0)),
            pl.BlockSpec((1, 1, T), lambda b: (b, 0, 0)),
        ),
        out_shape=(
            jax.ShapeDtypeStruct((B, N, T), jnp.float32),
            jax.ShapeDtypeStruct((B, 1, T), jnp.int32),
        ),
    )(hidden_states, Wq, bq2, Wp, bp2, keyflat)

    zq = _zq_gather(value_t, idx.reshape(B, T))

    return (logits, idx, zq)


# speedup vs baseline: 1.4302x; 1.2004x over previous
import functools
import math

import jax
import jax.numpy as jnp
from jax import lax
from jax.experimental import pallas as pl
from jax.experimental.pallas import tpu as pltpu
from jax.experimental.pallas import tpu_sc as plsc

B, C, T, N, H = 16, 512, 576, 1024, 4
DH = C // H
SF = 1.0 / math.sqrt(DH)
INV_SQRT_H = 1.0 / math.sqrt(H)

_NC = 2
_NS = 16
_L = 16
_NW = _NC * _NS
_ROWS = C // _NW
_NG = T // _L


def _bf16_round(x):
    return lax.convert_element_type(
        lax.convert_element_type(x, jnp.bfloat16), jnp.float32)


def _prep_body(cb_ref, wk_ref, bk_ref, wv_ref, bv_ref, kf_ref, vt_ref):
    cb = cb_ref[...]
    kf_ref[...] = lax.dot_general(
        cb, wk_ref[...], (((1,), (1,)), ((), ())),
        preferred_element_type=jnp.float32) + bk_ref[...]
    val = lax.dot_general(
        cb, wv_ref[...], (((1,), (1,)), ((), ())),
        preferred_element_type=jnp.float32) + bv_ref[...]
    vt_ref[...] = jnp.transpose(_bf16_round(val))


def _main_body(x_ref, wq_ref, bq_ref, wp_ref, bp_ref, kf_ref,
               logits_ref, idx_ref):
    x = x_ref[0]
    q = lax.dot_general(x, wq_ref[...], (((0,), (1,)), ((), ())),
                        preferred_element_type=jnp.float32)
    q = q + bq_ref[...]
    cp = lax.dot_general(x, wp_ref[...], (((0,), (1,)), ((), ())),
                         preferred_element_type=jnp.float32)
    cp = jnp.transpose(cp + bp_ref[...])
    cpb = _bf16_round(cp)
    kf = kf_ref[...]
    acc = None
    for h in range(H):
        q_h = q[:, h * DH:(h + 1) * DH]
        k_h = kf[:, h * DH:(h + 1) * DH]
        l1 = lax.dot_general(k_h, q_h, (((1,), (1,)), ((), ())),
                             preferred_element_type=jnp.float32) * SF
        term = cpb[h:h + 1, :] * _bf16_round(l1)
        acc = term if acc is None else acc + term
    logits = acc * INV_SQRT_H
    logits_ref[0] = logits
    maxv = jnp.max(logits, axis=0, keepdims=True)
    iota = lax.broadcasted_iota(jnp.int32, (N, T), 0)
    cand = jnp.where(logits == maxv, iota, N)
    idx = jnp.min(cand, axis=0, keepdims=True)
    idx_ref[0] = idx


def _zq_gather_body(vt_hbm, idx_hbm, zq_hbm, tbl_v, idx_v, out_v,
                    isem0, isem1, osem0, osem1):
    wid = lax.axis_index("s") * _NC + lax.axis_index("c")
    c0 = wid * _ROWS
    isems = (isem0, isem1)
    osems = (osem0, osem1)
    pltpu.sync_copy(vt_hbm.at[pl.ds(c0, _ROWS)], tbl_v)
    pltpu.async_copy(idx_hbm.at[0], idx_v.at[0], isems[0]).wait()
    out_cp = [None, None]
    for b in range(B):
        cur = b % 2
        nxt = (b + 1) % 2
        next_idx_cp = None
        if b + 1 < B:
            next_idx_cp = pltpu.async_copy(
                idx_hbm.at[b + 1], idx_v.at[nxt], isems[nxt])
        if out_cp[cur] is not None:
            out_cp[cur].wait()

        @plsc.parallel_loop(0, _NG, unroll=4)
        def gbody(g):
            ids = idx_v[cur, pl.ds(g * _L, _L)]
            for c in range(_ROWS):
                row = jnp.full((_L,), c, jnp.int32)
                out_v[cur, c, pl.ds(g * _L, _L)] = plsc.load_gather(
                    tbl_v, [row, ids])

        out_cp[cur] = pltpu.async_copy(
            out_v.at[cur], zq_hbm.at[b, pl.ds(c0, _ROWS)], osems[cur])
        if next_idx_cp is not None:
            next_idx_cp.wait()
    out_cp[0].wait()
    out_cp[1].wait()


def _zq_gather(value_t, idx2d):
    gather = functools.partial(
        pl.kernel,
        mesh=plsc.VectorSubcoreMesh(core_axis_name="c",
                                    subcore_axis_name="s"),
        compiler_params=pltpu.CompilerParams(needs_layout_passes=False),
        out_type=jax.ShapeDtypeStruct((B, C, T), jnp.float32),
        scratch_types=[
            pltpu.VMEM((_ROWS, N), jnp.float32),
            pltpu.VMEM((2, T), jnp.int32),
            pltpu.VMEM((2, _ROWS, T), jnp.float32),
            pltpu.SemaphoreType.DMA,
            pltpu.SemaphoreType.DMA,
            pltpu.SemaphoreType.DMA,
            pltpu.SemaphoreType.DMA,
        ],
    )(_zq_gather_body)
    return gather(value_t, idx2d)


def kernel(hidden_states, codebook_hidden_states, Wq, bq, Wk, bk, Wv, bv,
           Wp, bp):
    bk2 = bk.reshape(1, C)
    bv2 = bv.reshape(1, C)
    bq2 = bq.reshape(1, C)
    bp2 = bp.reshape(1, H)

    keyflat, value_t = pl.pallas_call(
        _prep_body,
        out_shape=(
            jax.ShapeDtypeStruct((N, C), jnp.float32),
            jax.ShapeDtypeStruct((C, N), jnp.float32),
        ),
    )(codebook_hidden_states, Wk, bk2, Wv, bv2)

    full = lambda shape: pl.BlockSpec(shape, lambda b: (0,) * len(shape))
    logits, idx = pl.pallas_call(
        _main_body,
        grid=(B,),
        in_specs=[
            pl.BlockSpec((1, C, T), lambda b: (b, 0, 0)),
            full((C, C)),
            full((1, C)),
            full((H, C)),
            full((1, H)),
            full((N, C)),
        ],
        out_specs=(
            pl.BlockSpec((1, N, T), lambda b: (b, 0, 0)),
            pl.BlockSpec((1, 1, T), lambda b: (b, 0, 0)),
        ),
        out_shape=(
            jax.ShapeDtypeStruct((B, N, T), jnp.float32),
            jax.ShapeDtypeStruct((B, 1, T), jnp.int32),
        ),
    )(hidden_states, Wq, bq2, Wp, bp2, keyflat)

    zq = _zq_gather(value_t, idx.reshape(B, T))

    return (logits, idx, zq)
